# baseline (speedup 1.0000x reference)
import functools

import jax
import jax.numpy as jnp
from jax.experimental import pallas as pl
from jax.experimental.pallas import tpu as pltpu

D_MODEL = 2048
D_OUT = 2048
E = 8
R = 64
ER = E * R
TOP_K = 2
SCALING = 128.0 / 64.0

TILE = 256


def _fused_kernel(xf_ref, w_ref, b_ref, rw_ref, a_ref, bcat_ref, o_ref):
    xf = xf_ref[...]
    xb = xf.astype(jnp.bfloat16)

    base = jax.lax.dot_general(
        xb, w_ref[...], (((1,), (1,)), ((), ())),
        preferred_element_type=jnp.float32)

    logits = jax.lax.dot_general(
        xf, rw_ref[...], (((1,), (0,)), ((), ())),
        precision=jax.lax.Precision.HIGHEST,
        preferred_element_type=jnp.float32)

    m = jnp.max(logits, axis=1, keepdims=True)
    p = jnp.exp(logits - m)
    eidx = jax.lax.broadcasted_iota(jnp.int32, (TILE, E), 1)

    v1 = jnp.max(p, axis=1, keepdims=True)
    i1 = jnp.min(jnp.where(p == v1, eidx, E), axis=1, keepdims=True)
    p2 = jnp.where(eidx == i1, -1.0, p)
    v2 = jnp.max(p2, axis=1, keepdims=True)
    i2 = jnp.min(jnp.where(p2 == v2, eidx, E), axis=1, keepdims=True)

    denom = v1 + v2
    g1 = (v1 / denom) * SCALING
    g2 = (v2 / denom) * SCALING

    h = jax.lax.dot_general(
        xb, a_ref[...], (((1,), (1,)), ((), ())),
        preferred_element_type=jnp.float32)

    ecol = jax.lax.broadcasted_iota(jnp.int32, (TILE, ER), 1) // R
    gates = jnp.where(ecol == i1, g1, 0.0) + jnp.where(ecol == i2, g2, 0.0)
    hw = (h * gates).astype(jnp.bfloat16)

    moe = jax.lax.dot_general(
        hw, bcat_ref[...], (((1,), (0,)), ((), ())),
        preferred_element_type=jnp.float32)

    o_ref[...] = base + moe + b_ref[...]


@functools.partial(jax.jit, static_argnames=())
def kernel(x, W_base, b_base, router_w, lora_A, lora_B):
    B, S, D = x.shape
    N = B * S
    xf = x.reshape(N, D)

    w_bf = W_base.astype(jnp.bfloat16)
    a_cat = lora_A.reshape(ER, D_MODEL).astype(jnp.bfloat16)
    b_cat = jnp.swapaxes(lora_B, 1, 2).reshape(ER, D_OUT).astype(jnp.bfloat16)
    b2 = b_base.reshape(1, D_OUT)

    grid = (N // TILE,)
    out = pl.pallas_call(
        _fused_kernel,
        grid=grid,
        in_specs=[
            pl.BlockSpec((TILE, D_MODEL), lambda i: (i, 0)),
            pl.BlockSpec((D_OUT, D_MODEL), lambda i: (0, 0)),
            pl.BlockSpec((1, D_OUT), lambda i: (0, 0)),
            pl.BlockSpec((D_MODEL, E), lambda i: (0, 0)),
            pl.BlockSpec((ER, D_MODEL), lambda i: (0, 0)),
            pl.BlockSpec((ER, D_OUT), lambda i: (0, 0)),
        ],
        out_specs=pl.BlockSpec((TILE, D_OUT), lambda i: (i, 0)),
        out_shape=jax.ShapeDtypeStruct((N, D_OUT), jnp.float32),
        compiler_params=pltpu.CompilerParams(
            dimension_semantics=("parallel",)),
    )(xf, w_bf, b2, router_w, a_cat, b_cat)
    return out.reshape(B, S, D_OUT)


# speedup vs baseline: 1.6092x; 1.6092x over previous
import functools

import jax
import jax.numpy as jnp
from jax.experimental import pallas as pl
from jax.experimental.pallas import tpu as pltpu

D_MODEL = 2048
D_OUT = 2048
E = 8
R = 64
ER = E * R
TOP_K = 2
SCALING = 128.0 / 64.0

TILE = 512


def _fused_kernel(xf_ref, w_ref, b_ref, rw_ref, a_ref, bcat_ref, o_ref):
    xf = xf_ref[...]
    xb = xf.astype(jnp.bfloat16)

    base = jax.lax.dot_general(
        xb, w_ref[...], (((1,), (1,)), ((), ())),
        preferred_element_type=jnp.float32)

    logits = jax.lax.dot_general(
        xb, rw_ref[...], (((1,), (0,)), ((), ())),
        preferred_element_type=jnp.float32)

    m = jnp.max(logits, axis=1, keepdims=True)
    p = jnp.exp(logits - m)
    eidx = jax.lax.broadcasted_iota(jnp.int32, (TILE, E), 1)

    v1 = jnp.max(p, axis=1, keepdims=True)
    i1 = jnp.min(jnp.where(p == v1, eidx, E), axis=1, keepdims=True)
    p2 = jnp.where(eidx == i1, -1.0, p)
    v2 = jnp.max(p2, axis=1, keepdims=True)
    i2 = jnp.min(jnp.where(p2 == v2, eidx, E), axis=1, keepdims=True)

    denom = v1 + v2
    g1 = (v1 / denom) * SCALING
    g2 = (v2 / denom) * SCALING

    h = jax.lax.dot_general(
        xb, a_ref[...], (((1,), (1,)), ((), ())),
        preferred_element_type=jnp.float32)

    ecol = jax.lax.broadcasted_iota(jnp.int32, (TILE, ER), 1) // R
    gates = jnp.where(ecol == i1, g1, 0.0) + jnp.where(ecol == i2, g2, 0.0)
    hw = (h * gates).astype(jnp.bfloat16)

    moe = jax.lax.dot_general(
        hw, bcat_ref[...], (((1,), (0,)), ((), ())),
        preferred_element_type=jnp.float32)

    o_ref[...] = base + moe + b_ref[...]


@functools.partial(jax.jit, static_argnames=())
def kernel(x, W_base, b_base, router_w, lora_A, lora_B):
    B, S, D = x.shape
    N = B * S
    xf = x.reshape(N, D)

    w_bf = W_base.astype(jnp.bfloat16)
    a_cat = lora_A.reshape(ER, D_MODEL).astype(jnp.bfloat16)
    b_cat = jnp.swapaxes(lora_B, 1, 2).reshape(ER, D_OUT).astype(jnp.bfloat16)
    b2 = b_base.reshape(1, D_OUT)
    rw_bf = router_w.astype(jnp.bfloat16)

    grid = (N // TILE,)
    out = pl.pallas_call(
        _fused_kernel,
        grid=grid,
        in_specs=[
            pl.BlockSpec((TILE, D_MODEL), lambda i: (i, 0)),
            pl.BlockSpec((D_OUT, D_MODEL), lambda i: (0, 0)),
            pl.BlockSpec((1, D_OUT), lambda i: (0, 0)),
            pl.BlockSpec((D_MODEL, E), lambda i: (0, 0)),
            pl.BlockSpec((ER, D_MODEL), lambda i: (0, 0)),
            pl.BlockSpec((ER, D_OUT), lambda i: (0, 0)),
        ],
        out_specs=pl.BlockSpec((TILE, D_OUT), lambda i: (i, 0)),
        out_shape=jax.ShapeDtypeStruct((N, D_OUT), jnp.float32),
        compiler_params=pltpu.CompilerParams(
            dimension_semantics=("parallel",)),
    )(xf, w_bf, b2, rw_bf, a_cat, b_cat)
    return out.reshape(B, S, D_OUT)


# speedup vs baseline: 1.8352x; 1.1405x over previous
import functools

import jax
import jax.numpy as jnp
from jax.experimental import pallas as pl
from jax.experimental.pallas import tpu as pltpu

D_MODEL = 2048
D_OUT = 2048
E = 8
R = 64
ER = E * R
SCALING = 128.0 / 64.0

TILE = 512
RW_PAD = 128
H_OFF = RW_PAD
B_OFF = RW_PAD + ER


def _fused_kernel(xf_ref, wcat_ref, b_ref, bcat_ref, o_ref):
    xb = xf_ref[...].astype(jnp.bfloat16)

    big = jax.lax.dot_general(
        xb, wcat_ref[...], (((1,), (1,)), ((), ())),
        preferred_element_type=jnp.float32)

    logits = big[:, :E]
    h = big[:, H_OFF:B_OFF]
    base = big[:, B_OFF:]

    m = jnp.max(logits, axis=1, keepdims=True)
    p = jnp.exp(logits - m)
    eidx = jax.lax.broadcasted_iota(jnp.int32, (TILE, E), 1)

    v1 = jnp.max(p, axis=1, keepdims=True)
    i1 = jnp.min(jnp.where(p == v1, eidx, E), axis=1, keepdims=True)
    p2 = jnp.where(eidx == i1, -1.0, p)
    v2 = jnp.max(p2, axis=1, keepdims=True)
    i2 = jnp.min(jnp.where(p2 == v2, eidx, E), axis=1, keepdims=True)

    denom = v1 + v2
    g1 = (v1 / denom) * SCALING
    g2 = (v2 / denom) * SCALING

    ecol = jax.lax.broadcasted_iota(jnp.int32, (TILE, ER), 1) // R
    gates = jnp.where(ecol == i1, g1, 0.0) + jnp.where(ecol == i2, g2, 0.0)
    hw = (h * gates).astype(jnp.bfloat16)

    moe = jax.lax.dot_general(
        hw, bcat_ref[...], (((1,), (0,)), ((), ())),
        preferred_element_type=jnp.float32)

    o_ref[...] = base + moe + b_ref[...]


@functools.partial(jax.jit, static_argnames=())
def kernel(x, W_base, b_base, router_w, lora_A, lora_B):
    B, S, D = x.shape
    N = B * S
    xf = x.reshape(N, D)

    rw_pad = jnp.zeros((RW_PAD, D_MODEL), jnp.bfloat16).at[:E].set(
        router_w.T.astype(jnp.bfloat16))
    w_cat = jnp.concatenate(
        [rw_pad,
         lora_A.reshape(ER, D_MODEL).astype(jnp.bfloat16),
         W_base.astype(jnp.bfloat16)], axis=0)
    b_cat = jnp.swapaxes(lora_B, 1, 2).reshape(ER, D_OUT).astype(jnp.bfloat16)
    b2 = b_base.reshape(1, D_OUT)

    grid = (N // TILE,)
    out = pl.pallas_call(
        _fused_kernel,
        grid=grid,
        in_specs=[
            pl.BlockSpec((TILE, D_MODEL), lambda i: (i, 0)),
            pl.BlockSpec((RW_PAD + ER + D_OUT, D_MODEL), lambda i: (0, 0)),
            pl.BlockSpec((1, D_OUT), lambda i: (0, 0)),
            pl.BlockSpec((ER, D_OUT), lambda i: (0, 0)),
        ],
        out_specs=pl.BlockSpec((TILE, D_OUT), lambda i: (i, 0)),
        out_shape=jax.ShapeDtypeStruct((N, D_OUT), jnp.float32),
        compiler_params=pltpu.CompilerParams(
            dimension_semantics=("arbitrary",)),
    )(xf, w_cat, b2, b_cat)
    return out.reshape(B, S, D_OUT)
